# baseline (speedup 1.0000x reference)
import functools

import jax
import jax.numpy as jnp
from jax import lax
from jax.experimental import pallas as pl
from jax.experimental.pallas import tpu as pltpu
from jax.experimental.pallas import tpu_sc as plsc

N_NODES = 10000
D = 128
E = 320000
NC = 2
NS = 16
NW = NC * NS
EPW = E // NW
CHUNK = 88
NFULL = EPW // CHUNK
TAIL = EPW - NFULL * CHUNK
NPAD = 10112
ROWS_PER_TILE = NPAD // NS
R = 4


def _sc_partial_sums(node, src, dst):
    mesh = plsc.VectorSubcoreMesh(core_axis_name="c", subcore_axis_name="s")

    @functools.partial(
        pl.kernel,
        out_type=jax.ShapeDtypeStruct((NC, NS, ROWS_PER_TILE, D), jnp.float32),
        mesh=mesh,
        scratch_types=(
            [pltpu.VMEM((CHUNK, D), jnp.float32) for _ in range(R)]
            + [pltpu.VMEM((CHUNK,), jnp.int32) for _ in range(2 * R)]
            + [pltpu.VMEM((TAIL,), jnp.int32) for _ in range(2)]
            + [pltpu.VMEM_SHARED((NPAD, D), jnp.float32)]
            + [pltpu.SemaphoreType.DMA for _ in range(2 * R)]
        ),
    )
    def body(node_hbm, src_hbm, dst_hbm, out_hbm,
             rows0, rows1, rows2, rows3,
             sx0, sx1, sx2, sx3, dx0, dx1, dx2, dx3, sxt, dxt,
             acc, g0, g1, g2, g3, s0, s1, s2, s3):
        rows = (rows0, rows1, rows2, rows3)
        sx = (sx0, sx1, sx2, sx3)
        dx = (dx0, dx1, dx2, dx3)
        gsem = (g0, g1, g2, g3)
        ssem = (s0, s1, s2, s3)

        c = lax.axis_index("c")
        s = lax.axis_index("s")
        w = c * NS + s

        @pl.loop(0, CHUNK)
        def _(r):
            for j in range(D // 16):
                rows0[r, pl.ds(j * 16, 16)] = jnp.zeros((16,), jnp.float32)

        for k in range(ROWS_PER_TILE // CHUNK):
            pltpu.sync_copy(
                rows0, acc.at[pl.ds(s * ROWS_PER_TILE + k * CHUNK, CHUNK)])
        zrem = ROWS_PER_TILE % CHUNK
        pltpu.sync_copy(
            rows0.at[pl.ds(0, zrem)],
            acc.at[pl.ds(s * ROWS_PER_TILE + ROWS_PER_TILE - zrem, zrem)])
        plsc.subcore_barrier()

        def stage_and_gather(i, b):
            base = w * EPW + i * CHUNK
            pltpu.sync_copy(src_hbm.at[pl.ds(base, CHUNK)], sx[b])
            pltpu.sync_copy(dst_hbm.at[pl.ds(base, CHUNK)], dx[b])
            pltpu.async_copy(node_hbm.at[sx[b]], rows[b], gsem[b])

        def wait_gather(b):
            pltpu.make_async_copy(node_hbm.at[sx[b]], rows[b], gsem[b]).wait()

        def start_scatter(b):
            pltpu.async_copy(rows[b], acc.at[dx[b]], ssem[b], add=True)

        def wait_scatter(b):
            pltpu.make_async_copy(rows[b], acc.at[dx[b]], ssem[b]).wait()

        for b in range(R - 1):
            stage_and_gather(b, b)

        @pl.loop(0, (NFULL - 5) // R)
        def _(t):
            for u in range(R):
                i = R * t + u
                b = u
                b3 = (u + 3) % R
                wait_gather(b)
                start_scatter(b)
                if u == 0:
                    @pl.when(t > 0)
                    def _():
                        wait_scatter(b3)
                else:
                    wait_scatter(b3)
                stage_and_gather(i + 3, b3)

        wait_gather(0)
        start_scatter(0)
        wait_scatter(3)
        stage_and_gather(NFULL - 2, 3)
        wait_gather(1)
        start_scatter(1)
        wait_scatter(0)
        stage_and_gather(NFULL - 1, 0)
        wait_gather(2)
        start_scatter(2)
        wait_gather(3)
        start_scatter(3)
        wait_gather(0)
        start_scatter(0)

        tbase = w * EPW + NFULL * CHUNK
        pltpu.sync_copy(src_hbm.at[pl.ds(tbase, TAIL)], sxt)
        pltpu.sync_copy(dst_hbm.at[pl.ds(tbase, TAIL)], dxt)
        wait_scatter(1)
        trows = rows1.at[pl.ds(0, TAIL)]
        pltpu.async_copy(node_hbm.at[sxt], trows, g1)
        pltpu.make_async_copy(node_hbm.at[sxt], trows, g1).wait()
        pltpu.async_copy(trows, acc.at[dxt], s1, add=True)
        pltpu.make_async_copy(trows, acc.at[dxt], s1).wait()
        wait_scatter(2)
        wait_scatter(3)
        wait_scatter(0)

        plsc.subcore_barrier()
        pltpu.sync_copy(
            acc.at[pl.ds(s * ROWS_PER_TILE, ROWS_PER_TILE)],
            out_hbm.at[c, s])

    return body(node, src, dst)


def _combine(partials):
    def body(p_ref, o_ref):
        o_ref[...] = p_ref[0, :N_NODES] + p_ref[1, :N_NODES]

    return pl.pallas_call(
        body,
        out_shape=jax.ShapeDtypeStruct((N_NODES, D), jnp.float32),
    )(partials)


@jax.jit
def kernel(node, edge_index):
    ei = edge_index.astype(jnp.int32)
    partials = _sc_partial_sums(node, ei[0], ei[1]).reshape(NC, NPAD, D)
    return _combine(partials)


# speedup vs baseline: 1.2311x; 1.2311x over previous
import functools

import jax
import jax.numpy as jnp
from jax import lax
from jax.experimental import pallas as pl
from jax.experimental.pallas import tpu as pltpu
from jax.experimental.pallas import tpu_sc as plsc

N_NODES = 10000
D = 128
E = 320000
NC = 2
NS = 16
NW = NC * NS
EPW = E // NW
CHUNK = 128
NFULL = EPW // CHUNK
TAIL = EPW - NFULL * CHUNK
NPAD = 10112
ROWS_PER_TILE = NPAD // NS
R = 3


def _sc_partial_sums(node, eidx):
    mesh = plsc.VectorSubcoreMesh(core_axis_name="c", subcore_axis_name="s")

    @functools.partial(
        pl.kernel,
        out_type=jax.ShapeDtypeStruct((NC, NS, ROWS_PER_TILE, D), jnp.float32),
        mesh=mesh,
        scratch_types=(
            [pltpu.VMEM((CHUNK, D), jnp.float32) for _ in range(R)]
            + [pltpu.VMEM((2, CHUNK), jnp.int32) for _ in range(R)]
            + [pltpu.VMEM((2, TAIL), jnp.int32)]
            + [pltpu.VMEM_SHARED((NPAD, D), jnp.float32)]
            + [pltpu.SemaphoreType.DMA for _ in range(2 * R)]
        ),
    )
    def body(node_hbm, eidx_hbm, out_hbm,
             rows0, rows1, rows2, ix0, ix1, ix2, ixt, acc,
             g0, g1, g2, s0, s1, s2):
        rows = (rows0, rows1, rows2)
        ix = (ix0, ix1, ix2)
        gsem = (g0, g1, g2)
        ssem = (s0, s1, s2)

        c = lax.axis_index("c")
        s = lax.axis_index("s")
        w = c * NS + s

        @pl.loop(0, CHUNK)
        def _(r):
            for j in range(D // 16):
                rows0[r, pl.ds(j * 16, 16)] = jnp.zeros((16,), jnp.float32)

        for k in range(ROWS_PER_TILE // CHUNK):
            pltpu.sync_copy(
                rows0, acc.at[pl.ds(s * ROWS_PER_TILE + k * CHUNK, CHUNK)])
        zrem = ROWS_PER_TILE % CHUNK
        pltpu.sync_copy(
            rows0.at[pl.ds(0, zrem)],
            acc.at[pl.ds(s * ROWS_PER_TILE + ROWS_PER_TILE - zrem, zrem)])
        plsc.subcore_barrier()

        def stage_and_gather(i, b):
            pltpu.sync_copy(eidx_hbm.at[w, :, pl.ds(i * CHUNK, CHUNK)], ix[b])
            pltpu.async_copy(node_hbm.at[ix[b].at[0]], rows[b], gsem[b])

        def wait_gather(b):
            pltpu.make_async_copy(node_hbm.at[ix[b].at[0]], rows[b],
                                  gsem[b]).wait()

        def start_scatter(b):
            pltpu.async_copy(rows[b], acc.at[ix[b].at[1]], ssem[b], add=True)

        def wait_scatter(b):
            pltpu.make_async_copy(rows[b], acc.at[ix[b].at[1]], ssem[b]).wait()

        stage_and_gather(0, 0)
        stage_and_gather(1, 1)

        @pl.loop(0, (NFULL - 3) // R)
        def _(t):
            for u in range(R):
                i = R * t + u
                b = u
                b2 = (u + 2) % R
                wait_gather(b)
                start_scatter(b)
                if u == 0:
                    @pl.when(t > 0)
                    def _():
                        wait_scatter(b2)
                else:
                    wait_scatter(b2)
                stage_and_gather(i + 2, b2)

        wait_gather(0)
        start_scatter(0)
        wait_scatter(2)
        stage_and_gather(NFULL - 1, 2)
        wait_gather(1)
        start_scatter(1)
        wait_gather(2)
        start_scatter(2)

        pltpu.sync_copy(eidx_hbm.at[w, :, pl.ds(NFULL * CHUNK, TAIL)], ixt)
        wait_scatter(0)
        trows = rows0.at[pl.ds(0, TAIL)]
        pltpu.async_copy(node_hbm.at[ixt.at[0]], trows, g0)
        pltpu.make_async_copy(node_hbm.at[ixt.at[0]], trows, g0).wait()
        pltpu.async_copy(trows, acc.at[ixt.at[1]], s0, add=True)
        pltpu.make_async_copy(trows, acc.at[ixt.at[1]], s0).wait()
        wait_scatter(1)
        wait_scatter(2)

        plsc.subcore_barrier()
        pltpu.sync_copy(
            acc.at[pl.ds(s * ROWS_PER_TILE, ROWS_PER_TILE)],
            out_hbm.at[c, s])

    return body(node, eidx)


def _combine(partials):
    def body(p_ref, o_ref):
        o_ref[...] = p_ref[0, :N_NODES] + p_ref[1, :N_NODES]

    return pl.pallas_call(
        body,
        out_shape=jax.ShapeDtypeStruct((N_NODES, D), jnp.float32),
    )(partials)


@jax.jit
def kernel(node, edge_index):
    eidx = edge_index.astype(jnp.int32).reshape(2, NW, EPW).transpose(1, 0, 2)
    partials = _sc_partial_sums(node, eidx).reshape(NC, NPAD, D)
    return _combine(partials)
